# baseline (speedup 1.0000x reference)
import functools

import jax
import jax.numpy as jnp
from jax import lax
from jax.experimental import pallas as pl
from jax.experimental.pallas import tpu as pltpu
from jax.experimental.pallas import tpu_sc as plsc

_VOCAB = 2051
_LANES = 16

_NC = 2
_NS = 16
_NW = _NC * _NS

_CHUNK_A = 32
_CHUNK_B = 24
_IDXW = 32


@functools.lru_cache(maxsize=None)
def _build(n_tokens: int, vocab_rows: int, d: int):
    per_w = n_tokens // _NW
    chunks = []
    off = 0
    use_a = True
    while off < per_w:
        size = _CHUNK_A if use_a else _CHUNK_B
        size = min(size, per_w - off)
        chunks.append((off, size, 0 if use_a else 1))
        off += size
        use_a = not use_a
    n_chunks = len(chunks)
    assert chunks[-1][0] + _IDXW <= per_w

    mesh = plsc.VectorSubcoreMesh(core_axis_name="c", subcore_axis_name="s")

    @functools.partial(
        pl.kernel,
        out_type=jax.ShapeDtypeStruct((n_tokens, d), jnp.float32),
        mesh=mesh,
        scratch_types=[
            pltpu.VMEM((per_w,), jnp.int32),
            pltpu.VMEM((per_w,), jnp.int32),
            pltpu.VMEM((n_chunks, _IDXW), jnp.int32),
            pltpu.VMEM((_CHUNK_A, d), jnp.float32),
            pltpu.VMEM((_CHUNK_B, d), jnp.float32),
            pltpu.SemaphoreType.DMA,
            pltpu.SemaphoreType.DMA,
            pltpu.SemaphoreType.DMA,
            pltpu.SemaphoreType.DMA,
        ],
    )
    def gather_kernel(ids_hbm, cbs_hbm, table_hbm, out_hbm,
                      ids_v, cbs_v, idx_v, buf_a, buf_b,
                      gsem_a, gsem_b, wsem_a, wsem_b):
        bufs = (buf_a, buf_b)
        gsems = (gsem_a, gsem_b)
        wsems = (wsem_a, wsem_b)
        wid = lax.axis_index("s") * _NC + lax.axis_index("c")
        base = wid * per_w

        pltpu.sync_copy(ids_hbm.at[pl.ds(base, per_w)], ids_v)
        pltpu.sync_copy(cbs_hbm.at[pl.ds(base, per_w)], cbs_v)

        def fill_idx(j):
            t0, size, _ = chunks[j]
            for p in range(0, size, _LANES):
                flat = (ids_v[pl.ds(t0 + p, _LANES)]
                        + cbs_v[pl.ds(t0 + p, _LANES)] * _VOCAB)
                idx_v[j, pl.ds(p, _LANES)] = flat

        def start_gather(j):
            _, size, b = chunks[j]
            return pltpu.async_copy(
                table_hbm.at[idx_v.at[j, pl.ds(0, size)]], bufs[b], gsems[b])

        def start_write(j):
            t0, size, b = chunks[j]
            return pltpu.async_copy(
                bufs[b], out_hbm.at[pl.ds(base + t0, size)], wsems[b])

        fill_idx(0)
        g0 = start_gather(0)
        fill_idx(1)
        g1 = start_gather(1)
        for j in range(2, n_chunks):
            fill_idx(j)

        g = [g0, g1] + [None] * (n_chunks - 2)
        w = [None] * n_chunks
        for k in range(n_chunks):
            g[k].wait()
            w[k] = start_write(k)
            if k + 2 < n_chunks:
                w[k].wait()
                g[k + 2] = start_gather(k + 2)
        for k in range(max(0, n_chunks - 2), n_chunks):
            w[k].wait()

    return gather_kernel


def kernel(input_ids, codebook_idxs, embed_audio_tokens_weight):
    b, s = input_ids.shape
    vocab_rows, d = embed_audio_tokens_weight.shape
    flat_ids = input_ids.reshape(-1).astype(jnp.int32)
    flat_cbs = codebook_idxs.reshape(-1).astype(jnp.int32)
    out = _build(b * s, vocab_rows, d)(flat_ids, flat_cbs, embed_audio_tokens_weight)
    return out.reshape(b, s, d)
